# baseline (speedup 1.0000x reference)
import functools

import jax
import jax.numpy as jnp
from jax import lax
from jax.experimental import pallas as pl
from jax.experimental.pallas import tpu as pltpu
from jax.experimental.pallas import tpu_sc as plsc

N = 10000
E = 320000
D = 128
H = 64
G = 100

NC = 2
NS = 16
NW = NC * NS
CHUNK = 128

EPS = E // NW
NCHUNK = 128
EPT = NCHUNK * CHUNK
MINP = -(-EPS // CHUNK)
RPT = 640
R = NS * RPT
PRIV = N
SINK = R - 1

PCHUNK = -(-N // (NW * CHUNK))
NPAD = NW * PCHUNK * CHUNK
PRPT = 8
PR = NS * PRPT
PSINK = 127

ROWS = 1000
GRID = N // ROWS


def _sc_scatter_body(nchunk, rpt, width, u_hbm, srcs_hbm, dsts_hbm, out_hbm,
                     src_v, dst_v, rows_v, zbuf, acc, sem):
    c = lax.axis_index("c")
    s = lax.axis_index("s")
    wid = c * NS + s
    zrows = min(rpt, CHUNK)

    def _zrow(i, _):
        for j in range(width // 16):
            zbuf[i, pl.ds(j * 16, 16)] = jnp.zeros((16,), jnp.float32)
        return 0
    lax.fori_loop(0, zrows, _zrow, 0)

    def _zcp(k, _):
        pltpu.sync_copy(zbuf.at[pl.ds(0, zrows)],
                        acc.at[pl.ds(s * rpt + k * zrows, zrows)])
        return 0
    lax.fori_loop(0, rpt // zrows, _zcp, 0)
    plsc.subcore_barrier()

    pltpu.sync_copy(srcs_hbm.at[wid], src_v)
    pltpu.sync_copy(dsts_hbm.at[wid], dst_v)

    def _chunk(j, _):
        pltpu.async_copy(u_hbm.at[src_v.at[j]], rows_v, sem).wait()
        pltpu.sync_copy(rows_v, acc.at[dst_v.at[j]], add=True)
        return 0
    lax.fori_loop(0, nchunk, _chunk, 0)
    plsc.subcore_barrier()

    def _rb(k, _):
        base = s * rpt + k * zrows
        pltpu.sync_copy(acc.at[pl.ds(base, zrows)],
                        rows_v.at[pl.ds(0, zrows)])
        pltpu.sync_copy(rows_v.at[pl.ds(0, zrows)],
                        out_hbm.at[c, pl.ds(base, zrows)])
        return 0
    lax.fori_loop(0, rpt // zrows, _rb, 0)


def _make_sc_scatter(nchunk, rpt, width, acc_rows):
    return pl.kernel(
        functools.partial(_sc_scatter_body, nchunk, rpt, width),
        out_type=jax.ShapeDtypeStruct((NC, acc_rows, width), jnp.float32),
        mesh=plsc.VectorSubcoreMesh(core_axis_name="c", subcore_axis_name="s"),
        scratch_types=[
            pltpu.VMEM((nchunk, CHUNK), jnp.int32),
            pltpu.VMEM((nchunk, CHUNK), jnp.int32),
            pltpu.VMEM((CHUNK, width), jnp.float32),
            pltpu.VMEM((min(rpt, CHUNK), width), jnp.float32),
            pltpu.VMEM_SHARED((acc_rows, width), jnp.float32),
            pltpu.SemaphoreType.DMA,
        ],
        compiler_params=pltpu.CompilerParams(use_tc_tiling_on_sc=False),
    )


_sc_agg = _make_sc_scatter(NCHUNK, RPT, H, R)
_sc_pool = _make_sc_scatter(PCHUNK, PRPT, H, PR)


def _layer0_body(h_ref, pa_ref, pb_ref, w1_ref, b1_ref, w2_ref, b2_ref,
                 o_ref):
    agg = jnp.concatenate([pa_ref[...], pb_ref[...]], axis=1)
    m = h_ref[...] + agg
    t = jnp.maximum(
        jnp.dot(m, w1_ref[...], preferred_element_type=jnp.float32)
        + b1_ref[...], 0.0)
    o_ref[...] = jnp.maximum(
        jnp.dot(t, w2_ref[...], preferred_element_type=jnp.float32)
        + b2_ref[...], 0.0)


def _layer0(h, pa, pb, w1, b1, w2, b2):
    return pl.pallas_call(
        _layer0_body,
        grid=(GRID,),
        in_specs=[pl.BlockSpec((ROWS, D), lambda i: (i, 0)),
                  pl.BlockSpec((ROWS, H), lambda i: (i, 0)),
                  pl.BlockSpec((ROWS, H), lambda i: (i, 0)),
                  pl.BlockSpec((D, H), lambda i: (0, 0)),
                  pl.BlockSpec((1, H), lambda i: (0, 0)),
                  pl.BlockSpec((H, H), lambda i: (0, 0)),
                  pl.BlockSpec((1, H), lambda i: (0, 0))],
        out_specs=pl.BlockSpec((ROWS, H), lambda i: (i, 0)),
        out_shape=jax.ShapeDtypeStruct((N, H), jnp.float32),
    )(h, pa, pb, w1, b1, w2, b2)


def _layer_body(h_ref, a_ref, w1_ref, b1_ref, w2_ref, b2_ref, o_ref):
    m = h_ref[...] + a_ref[...]
    t = jnp.maximum(
        jnp.dot(m, w1_ref[...], preferred_element_type=jnp.float32)
        + b1_ref[...], 0.0)
    o_ref[...] = jnp.maximum(
        jnp.dot(t, w2_ref[...], preferred_element_type=jnp.float32)
        + b2_ref[...], 0.0)


def _layer(h, a, w1, b1, w2, b2):
    return pl.pallas_call(
        _layer_body,
        grid=(GRID,),
        in_specs=[pl.BlockSpec((ROWS, H), lambda i: (i, 0)),
                  pl.BlockSpec((ROWS, H), lambda i: (i, 0)),
                  pl.BlockSpec((H, H), lambda i: (0, 0)),
                  pl.BlockSpec((1, H), lambda i: (0, 0)),
                  pl.BlockSpec((H, H), lambda i: (0, 0)),
                  pl.BlockSpec((1, H), lambda i: (0, 0))],
        out_specs=pl.BlockSpec((ROWS, H), lambda i: (i, 0)),
        out_shape=jax.ShapeDtypeStruct((N, H), jnp.float32),
    )(h, a, w1, b1, w2, b2)


def _jk_body(h1_ref, h2_ref, h3_ref, wjk_ref, bjk_ref, o_ref):
    hcat = jnp.concatenate([h1_ref[...], h2_ref[...], h3_ref[...]], axis=1)
    o_ref[...] = (jnp.dot(hcat, wjk_ref[...],
                          preferred_element_type=jnp.float32)
                  + bjk_ref[...])


def _jk(h1, h2, h3, wjk, bjk):
    return pl.pallas_call(
        _jk_body,
        grid=(GRID,),
        in_specs=[pl.BlockSpec((ROWS, H), lambda i: (i, 0)),
                  pl.BlockSpec((ROWS, H), lambda i: (i, 0)),
                  pl.BlockSpec((ROWS, H), lambda i: (i, 0)),
                  pl.BlockSpec((3 * H, H), lambda i: (0, 0)),
                  pl.BlockSpec((1, H), lambda i: (0, 0))],
        out_specs=pl.BlockSpec((ROWS, H), lambda i: (i, 0)),
        out_shape=jax.ShapeDtypeStruct((N, H), jnp.float32),
    )(h1, h2, h3, wjk, bjk)


def _cls_body(p_ref, wc1_ref, bc1_ref, g_ref, be_ref, wc2_ref, bc2_ref,
              o_ref):
    pooled = p_ref[0] + p_ref[1]
    z = (jnp.dot(pooled, wc1_ref[...], preferred_element_type=jnp.float32)
         + bc1_ref[...])
    rmask = lax.broadcasted_iota(jnp.int32, (PR, H), 0) < G
    zm = jnp.where(rmask, z, 0.0)
    mean = jnp.sum(zm, axis=0, keepdims=True) / G
    zc = jnp.where(rmask, z - mean, 0.0)
    var = jnp.sum(zc * zc, axis=0, keepdims=True) / G
    zn = (z - mean) / jnp.sqrt(var + 1e-5) * g_ref[...] + be_ref[...]
    zn = jnp.maximum(zn, 0.0)
    o_ref[...] = (jnp.dot(zn, wc2_ref[...], preferred_element_type=jnp.float32)
                  + bc2_ref[...])


def _cls(p, wc1, bc1, gamma, beta, wc2, bc2):
    return pl.pallas_call(
        _cls_body,
        in_specs=[pl.BlockSpec((NC, PR, H), lambda: (0, 0, 0)),
                  pl.BlockSpec((H, H), lambda: (0, 0)),
                  pl.BlockSpec((1, H), lambda: (0, 0)),
                  pl.BlockSpec((1, H), lambda: (0, 0)),
                  pl.BlockSpec((1, H), lambda: (0, 0)),
                  pl.BlockSpec((H, H), lambda: (0, 0)),
                  pl.BlockSpec((1, H), lambda: (0, 0))],
        out_specs=pl.BlockSpec((PR, H), lambda: (0, 0)),
        out_shape=jax.ShapeDtypeStruct((PR, H), jnp.float32),
    )(p, wc1, bc1, gamma, beta, wc2, bc2)


def _edge_slabs(src, dst):
    order = jnp.argsort(dst, stable=True)
    sd = dst[order].reshape(NW, EPS)
    ss = src[order].reshape(NW, EPS)

    firsts = sd[:, 0]
    crossing = jnp.concatenate(
        [jnp.zeros((1,), bool), firsts[1:] == sd[:-1, -1]])
    tid = jnp.arange(NW, dtype=sd.dtype)
    sdr = jnp.where(crossing[:, None] & (sd == firsts[:, None]),
                    PRIV + tid[:, None], sd)
    bd = jnp.where(crossing, firsts, SINK)

    fo = jax.vmap(
        lambda row: jnp.searchsorted(row, row, side='left'))(sd)

    def step(b, _):
        cand = jnp.minimum(b + NCHUNK, EPS)
        fc = jnp.where(
            cand >= EPS, EPS,
            jnp.take_along_axis(
                fo, jnp.minimum(cand, EPS - 1)[:, None], axis=1)[:, 0])
        nb = jnp.maximum(fc, jnp.minimum(b + MINP, EPS))
        return nb, nb

    _, bs = lax.scan(step, jnp.zeros((NW,), jnp.int32), None, length=CHUNK)
    bs = jnp.concatenate([jnp.zeros((1, NW), jnp.int32), bs])
    bsT = bs.T
    j = jnp.arange(EPS, dtype=jnp.int32)
    seg = jax.vmap(
        lambda b: jnp.searchsorted(b, j, side='right') - 1)(bsT)
    off = j[None, :] - jnp.take_along_axis(bsT, seg, axis=1)
    pos = (tid[:, None] * EPT + off * CHUNK + seg).reshape(-1)

    grid_d = jnp.full((NW * EPT,), SINK, sd.dtype).at[pos].set(sdr.reshape(-1))
    grid_s = (jnp.arange(NW * EPT, dtype=sd.dtype) % N).at[pos].set(
        ss.reshape(-1))
    srcp = grid_s.reshape(NW, NCHUNK, CHUNK)
    dstp = grid_d.reshape(NW, NCHUNK, CHUNK)
    return srcp, dstp, bd


def kernel(x, W1_0, b1_0, W2_0, b2_0, W1_1, b1_1, W2_1, b2_1,
           W1_2, b1_2, W2_2, b2_2, Wjk, bjk, Wc1, bc1, gamma, beta,
           Wc2, bc2, edge_index, batch):
    srcp, dstp, bd = _edge_slabs(edge_index[0], edge_index[1])

    def merge(p):
        a = p[0] + p[1]
        priv = lax.slice(a, (PRIV, 0), (PRIV + NW, a.shape[1]))
        return a.at[bd].add(priv)

    nodep = jnp.pad(jnp.arange(N, dtype=jnp.int32),
                    (0, NPAD - N)).reshape(NW, PCHUNK, CHUNK)
    batchp = jnp.pad(batch, (0, NPAD - N),
                     constant_values=PSINK).reshape(NW, PCHUNK, CHUNK)
    r = lambda b: b.reshape(1, -1)

    xa = lax.slice(x, (0, 0), (N, H))
    xb = lax.slice(x, (0, H), (N, D))
    a0a = merge(_sc_agg(xa, srcp, dstp))
    a0b = merge(_sc_agg(xb, srcp, dstp))
    h1 = _layer0(x, a0a, a0b, W1_0, r(b1_0), W2_0, r(b2_0))
    a1 = merge(_sc_agg(h1, srcp, dstp))
    h2 = _layer(h1, a1, W1_1, r(b1_1), W2_1, r(b2_1))
    a2 = merge(_sc_agg(h2, srcp, dstp))
    h3 = _layer(h2, a2, W1_2, r(b1_2), W2_2, r(b2_2))
    hjk = _jk(h1, h2, h3, Wjk, r(bjk))
    p = _sc_pool(hjk, nodep, batchp)
    out = _cls(p, Wc1, r(bc1), r(gamma), r(beta), Wc2, r(bc2))
    return out[:G]


# speedup vs baseline: 1.0004x; 1.0004x over previous
import functools

import jax
import jax.numpy as jnp
from jax import lax
from jax.experimental import pallas as pl
from jax.experimental.pallas import tpu as pltpu
from jax.experimental.pallas import tpu_sc as plsc

N = 10000
E = 320000
D = 128
H = 64
G = 100

NC = 2
NS = 16
NW = NC * NS
CHUNK = 128

EPS = E // NW
NCHUNK = 128
EPT = NCHUNK * CHUNK
MINP = -(-EPS // CHUNK)
RPT = 640
R = NS * RPT
PRIV = N
SINK = R - 1

PCHUNK = -(-N // (NW * CHUNK))
NPAD = NW * PCHUNK * CHUNK
PRPT = 8
PR = NS * PRPT
PSINK = 127

ROWS = 1000
GRID = N // ROWS


def _sc_scatter_body(nchunk, rpt, width, u_hbm, srcs_hbm, dsts_hbm, out_hbm,
                     src_v, dst_v, rows_v, zbuf, acc, sem):
    c = lax.axis_index("c")
    s = lax.axis_index("s")
    wid = c * NS + s
    zrows = min(rpt, CHUNK)

    def _zrow(i, _):
        for j in range(width // 16):
            zbuf[i, pl.ds(j * 16, 16)] = jnp.zeros((16,), jnp.float32)
        return 0
    lax.fori_loop(0, zrows, _zrow, 0)

    def _zcp(k, _):
        pltpu.sync_copy(zbuf.at[pl.ds(0, zrows)],
                        acc.at[pl.ds(s * rpt + k * zrows, zrows)])
        return 0
    lax.fori_loop(0, rpt // zrows, _zcp, 0)
    plsc.subcore_barrier()

    pltpu.sync_copy(srcs_hbm.at[wid], src_v)
    pltpu.sync_copy(dsts_hbm.at[wid], dst_v)

    def _chunk(j, _):
        pltpu.async_copy(u_hbm.at[src_v.at[j]], rows_v, sem).wait()
        pltpu.sync_copy(rows_v, acc.at[dst_v.at[j]], add=True)
        return 0
    lax.fori_loop(0, nchunk, _chunk, 0)
    plsc.subcore_barrier()

    def _rb(k, _):
        base = s * rpt + k * zrows
        pltpu.sync_copy(acc.at[pl.ds(base, zrows)],
                        rows_v.at[pl.ds(0, zrows)])
        pltpu.sync_copy(rows_v.at[pl.ds(0, zrows)],
                        out_hbm.at[c, pl.ds(base, zrows)])
        return 0
    lax.fori_loop(0, rpt // zrows, _rb, 0)


def _make_sc_scatter(nchunk, rpt, width, acc_rows):
    return pl.kernel(
        functools.partial(_sc_scatter_body, nchunk, rpt, width),
        out_type=jax.ShapeDtypeStruct((NC, acc_rows, width), jnp.float32),
        mesh=plsc.VectorSubcoreMesh(core_axis_name="c", subcore_axis_name="s"),
        scratch_types=[
            pltpu.VMEM((nchunk, CHUNK), jnp.int32),
            pltpu.VMEM((nchunk, CHUNK), jnp.int32),
            pltpu.VMEM((CHUNK, width), jnp.float32),
            pltpu.VMEM((min(rpt, CHUNK), width), jnp.float32),
            pltpu.VMEM_SHARED((acc_rows, width), jnp.float32),
            pltpu.SemaphoreType.DMA,
        ],
        compiler_params=pltpu.CompilerParams(use_tc_tiling_on_sc=False),
    )


_sc_agg = _make_sc_scatter(NCHUNK, RPT, H, R)
_sc_pool = _make_sc_scatter(PCHUNK, PRPT, H, PR)


def _layer0_body(h_ref, pa_ref, pb_ref, w1_ref, b1_ref, w2_ref, b2_ref,
                 o_ref):
    agg = jnp.concatenate([pa_ref[...], pb_ref[...]], axis=1)
    m = h_ref[...] + agg
    t = jnp.maximum(
        jnp.dot(m, w1_ref[...], preferred_element_type=jnp.float32)
        + b1_ref[...], 0.0)
    o_ref[...] = jnp.maximum(
        jnp.dot(t, w2_ref[...], preferred_element_type=jnp.float32)
        + b2_ref[...], 0.0)


def _layer0(h, pa, pb, w1, b1, w2, b2):
    return pl.pallas_call(
        _layer0_body,
        grid=(GRID,),
        in_specs=[pl.BlockSpec((ROWS, D), lambda i: (i, 0)),
                  pl.BlockSpec((ROWS, H), lambda i: (i, 0)),
                  pl.BlockSpec((ROWS, H), lambda i: (i, 0)),
                  pl.BlockSpec((D, H), lambda i: (0, 0)),
                  pl.BlockSpec((1, H), lambda i: (0, 0)),
                  pl.BlockSpec((H, H), lambda i: (0, 0)),
                  pl.BlockSpec((1, H), lambda i: (0, 0))],
        out_specs=pl.BlockSpec((ROWS, H), lambda i: (i, 0)),
        out_shape=jax.ShapeDtypeStruct((N, H), jnp.float32),
    )(h, pa, pb, w1, b1, w2, b2)


def _layer_body(h_ref, a_ref, w1_ref, b1_ref, w2_ref, b2_ref, o_ref):
    m = h_ref[...] + a_ref[...]
    t = jnp.maximum(
        jnp.dot(m, w1_ref[...], preferred_element_type=jnp.float32)
        + b1_ref[...], 0.0)
    o_ref[...] = jnp.maximum(
        jnp.dot(t, w2_ref[...], preferred_element_type=jnp.float32)
        + b2_ref[...], 0.0)


def _layer(h, a, w1, b1, w2, b2):
    return pl.pallas_call(
        _layer_body,
        grid=(GRID,),
        in_specs=[pl.BlockSpec((ROWS, H), lambda i: (i, 0)),
                  pl.BlockSpec((ROWS, H), lambda i: (i, 0)),
                  pl.BlockSpec((H, H), lambda i: (0, 0)),
                  pl.BlockSpec((1, H), lambda i: (0, 0)),
                  pl.BlockSpec((H, H), lambda i: (0, 0)),
                  pl.BlockSpec((1, H), lambda i: (0, 0))],
        out_specs=pl.BlockSpec((ROWS, H), lambda i: (i, 0)),
        out_shape=jax.ShapeDtypeStruct((N, H), jnp.float32),
    )(h, a, w1, b1, w2, b2)


def _jk_body(h1_ref, h2_ref, h3_ref, wjk_ref, bjk_ref, o_ref):
    hcat = jnp.concatenate([h1_ref[...], h2_ref[...], h3_ref[...]], axis=1)
    o_ref[...] = (jnp.dot(hcat, wjk_ref[...],
                          preferred_element_type=jnp.float32)
                  + bjk_ref[...])


def _jk(h1, h2, h3, wjk, bjk):
    return pl.pallas_call(
        _jk_body,
        grid=(GRID,),
        in_specs=[pl.BlockSpec((ROWS, H), lambda i: (i, 0)),
                  pl.BlockSpec((ROWS, H), lambda i: (i, 0)),
                  pl.BlockSpec((ROWS, H), lambda i: (i, 0)),
                  pl.BlockSpec((3 * H, H), lambda i: (0, 0)),
                  pl.BlockSpec((1, H), lambda i: (0, 0))],
        out_specs=pl.BlockSpec((ROWS, H), lambda i: (i, 0)),
        out_shape=jax.ShapeDtypeStruct((N, H), jnp.float32),
    )(h1, h2, h3, wjk, bjk)


def _cls_body(p_ref, wc1_ref, bc1_ref, g_ref, be_ref, wc2_ref, bc2_ref,
              o_ref):
    pooled = p_ref[0] + p_ref[1]
    z = (jnp.dot(pooled, wc1_ref[...], preferred_element_type=jnp.float32)
         + bc1_ref[...])
    rmask = lax.broadcasted_iota(jnp.int32, (PR, H), 0) < G
    zm = jnp.where(rmask, z, 0.0)
    mean = jnp.sum(zm, axis=0, keepdims=True) / G
    zc = jnp.where(rmask, z - mean, 0.0)
    var = jnp.sum(zc * zc, axis=0, keepdims=True) / G
    zn = (z - mean) / jnp.sqrt(var + 1e-5) * g_ref[...] + be_ref[...]
    zn = jnp.maximum(zn, 0.0)
    o_ref[...] = (jnp.dot(zn, wc2_ref[...], preferred_element_type=jnp.float32)
                  + bc2_ref[...])


def _cls(p, wc1, bc1, gamma, beta, wc2, bc2):
    return pl.pallas_call(
        _cls_body,
        in_specs=[pl.BlockSpec((NC, PR, H), lambda: (0, 0, 0)),
                  pl.BlockSpec((H, H), lambda: (0, 0)),
                  pl.BlockSpec((1, H), lambda: (0, 0)),
                  pl.BlockSpec((1, H), lambda: (0, 0)),
                  pl.BlockSpec((1, H), lambda: (0, 0)),
                  pl.BlockSpec((H, H), lambda: (0, 0)),
                  pl.BlockSpec((1, H), lambda: (0, 0))],
        out_specs=pl.BlockSpec((PR, H), lambda: (0, 0)),
        out_shape=jax.ShapeDtypeStruct((PR, H), jnp.float32),
    )(p, wc1, bc1, gamma, beta, wc2, bc2)


def _edge_slabs(src, dst):
    order = jnp.argsort(dst, stable=True)
    sd = dst[order].reshape(NW, EPS)
    ss = src[order].reshape(NW, EPS)

    firsts = sd[:, 0]
    crossing = jnp.concatenate(
        [jnp.zeros((1,), bool), firsts[1:] == sd[:-1, -1]])
    tid = jnp.arange(NW, dtype=sd.dtype)
    sdr = jnp.where(crossing[:, None] & (sd == firsts[:, None]),
                    PRIV + tid[:, None], sd)
    bd = jnp.where(crossing, firsts, SINK)

    fo = jax.vmap(
        lambda row: jnp.searchsorted(row, row, side='left'))(sd)

    def step(b, _):
        cand = jnp.minimum(b + NCHUNK, EPS)
        fc = jnp.where(
            cand >= EPS, EPS,
            jnp.take_along_axis(
                fo, jnp.minimum(cand, EPS - 1)[:, None], axis=1)[:, 0])
        nb = jnp.maximum(fc, jnp.minimum(b + MINP, EPS))
        return nb, nb

    _, bs = lax.scan(step, jnp.zeros((NW,), jnp.int32), None, length=CHUNK)
    bs = jnp.concatenate([jnp.zeros((1, NW), jnp.int32), bs])
    bsT = bs.T
    j = jnp.arange(EPS, dtype=jnp.int32)
    seg = jax.vmap(
        lambda b: jnp.searchsorted(b, j, side='right') - 1)(bsT)
    off = j[None, :] - jnp.take_along_axis(bsT, seg, axis=1)
    pos = (tid[:, None] * EPT + off * CHUNK + seg).reshape(-1)

    pad_rows = PRIV + NW + 32 + jnp.arange(NW * EPT, dtype=sd.dtype) % 64
    grid_d = pad_rows.at[pos].set(sdr.reshape(-1))
    grid_s = (jnp.arange(NW * EPT, dtype=sd.dtype) % N).at[pos].set(
        ss.reshape(-1))
    srcp = grid_s.reshape(NW, NCHUNK, CHUNK)
    dstp = grid_d.reshape(NW, NCHUNK, CHUNK)
    return srcp, dstp, bd


def kernel(x, W1_0, b1_0, W2_0, b2_0, W1_1, b1_1, W2_1, b2_1,
           W1_2, b1_2, W2_2, b2_2, Wjk, bjk, Wc1, bc1, gamma, beta,
           Wc2, bc2, edge_index, batch):
    srcp, dstp, bd = _edge_slabs(edge_index[0], edge_index[1])

    def merge(p):
        a = p[0] + p[1]
        priv = lax.slice(a, (PRIV, 0), (PRIV + NW, a.shape[1]))
        return a.at[bd].add(priv)

    nodep = jnp.pad(jnp.arange(N, dtype=jnp.int32),
                    (0, NPAD - N)).reshape(NW, PCHUNK, CHUNK)
    batchp = jnp.pad(batch, (0, NPAD - N),
                     constant_values=PSINK).reshape(NW, PCHUNK, CHUNK)
    r = lambda b: b.reshape(1, -1)

    xa = lax.slice(x, (0, 0), (N, H))
    xb = lax.slice(x, (0, H), (N, D))
    a0a = merge(_sc_agg(xa, srcp, dstp))
    a0b = merge(_sc_agg(xb, srcp, dstp))
    h1 = _layer0(x, a0a, a0b, W1_0, r(b1_0), W2_0, r(b2_0))
    a1 = merge(_sc_agg(h1, srcp, dstp))
    h2 = _layer(h1, a1, W1_1, r(b1_1), W2_1, r(b2_1))
    a2 = merge(_sc_agg(h2, srcp, dstp))
    h3 = _layer(h2, a2, W1_2, r(b1_2), W2_2, r(b2_2))
    hjk = _jk(h1, h2, h3, Wjk, r(bjk))
    p = _sc_pool(hjk, nodep, batchp)
    out = _cls(p, Wc1, r(bc1), r(gamma), r(beta), Wc2, r(bc2))
    return out[:G]


# speedup vs baseline: 20.4700x; 20.4619x over previous
import functools

import jax
import jax.numpy as jnp
from jax import lax
from jax.experimental import pallas as pl
from jax.experimental.pallas import tpu as pltpu
from jax.experimental.pallas import tpu_sc as plsc

N = 10000
E = 320000
D = 128
H = 64
G = 100

NC = 2
NS = 16
NW = NC * NS
CHUNK = 128

EPS = E // NW
NCHUNK = 128
EPT = NCHUNK * CHUNK
MINP = -(-EPS // CHUNK)
RPT = 640
R = NS * RPT
PRIV = N
SINK = R - 1

PCHUNK = -(-N // (NW * CHUNK))
NPAD = NW * PCHUNK * CHUNK
PRPT = 8
PR = NS * PRPT
PSINK = 127

ROWS = 1000
GRID = N // ROWS


def _sc_scatter_body(nchunk, rpt, width, u_hbm, srcs_hbm, dsts_hbm, out_hbm,
                     src_v, dst_v, rows_v, zbuf, acc, sem):
    c = lax.axis_index("c")
    s = lax.axis_index("s")
    wid = c * NS + s
    zrows = min(rpt, CHUNK)

    def _zrow(i, _):
        for j in range(width // 16):
            zbuf[i, pl.ds(j * 16, 16)] = jnp.zeros((16,), jnp.float32)
        return 0
    lax.fori_loop(0, zrows, _zrow, 0)

    def _zcp(k, _):
        pltpu.sync_copy(zbuf.at[pl.ds(0, zrows)],
                        acc.at[pl.ds(s * rpt + k * zrows, zrows)])
        return 0
    lax.fori_loop(0, rpt // zrows, _zcp, 0)
    plsc.subcore_barrier()

    pltpu.sync_copy(srcs_hbm.at[wid], src_v)
    pltpu.sync_copy(dsts_hbm.at[wid], dst_v)

    def _chunk(j, _):
        pltpu.async_copy(u_hbm.at[src_v.at[j]], rows_v, sem).wait()
        pltpu.sync_copy(rows_v, acc.at[dst_v.at[j]], add=True)
        return 0
    lax.fori_loop(0, nchunk, _chunk, 0)
    plsc.subcore_barrier()

    def _rb(k, _):
        base = s * rpt + k * zrows
        pltpu.sync_copy(acc.at[pl.ds(base, zrows)],
                        rows_v.at[pl.ds(0, zrows)])
        pltpu.sync_copy(rows_v.at[pl.ds(0, zrows)],
                        out_hbm.at[c, pl.ds(base, zrows)])
        return 0
    lax.fori_loop(0, rpt // zrows, _rb, 0)


def _make_sc_scatter(nchunk, rpt, width, acc_rows):
    return pl.kernel(
        functools.partial(_sc_scatter_body, nchunk, rpt, width),
        out_type=jax.ShapeDtypeStruct((NC, acc_rows, width), jnp.float32),
        mesh=plsc.VectorSubcoreMesh(core_axis_name="c", subcore_axis_name="s"),
        scratch_types=[
            pltpu.VMEM((nchunk, CHUNK), jnp.int32),
            pltpu.VMEM((nchunk, CHUNK), jnp.int32),
            pltpu.VMEM((CHUNK, width), jnp.float32),
            pltpu.VMEM((min(rpt, CHUNK), width), jnp.float32),
            pltpu.VMEM_SHARED((acc_rows, width), jnp.float32),
            pltpu.SemaphoreType.DMA,
        ],
        compiler_params=pltpu.CompilerParams(use_tc_tiling_on_sc=False),
    )


_sc_agg = _make_sc_scatter(NCHUNK, RPT, H, R)
_sc_pool = _make_sc_scatter(PCHUNK, PRPT, H, PR)


def _layer0_body(h_ref, pa_ref, pb_ref, w1_ref, b1_ref, w2_ref, b2_ref,
                 o_ref):
    agg = jnp.concatenate([pa_ref[...], pb_ref[...]], axis=1)
    m = h_ref[...] + agg
    t = jnp.maximum(
        jnp.dot(m, w1_ref[...], preferred_element_type=jnp.float32)
        + b1_ref[...], 0.0)
    o_ref[...] = jnp.maximum(
        jnp.dot(t, w2_ref[...], preferred_element_type=jnp.float32)
        + b2_ref[...], 0.0)


def _layer0(h, pa, pb, w1, b1, w2, b2):
    return pl.pallas_call(
        _layer0_body,
        grid=(GRID,),
        in_specs=[pl.BlockSpec((ROWS, D), lambda i: (i, 0)),
                  pl.BlockSpec((ROWS, H), lambda i: (i, 0)),
                  pl.BlockSpec((ROWS, H), lambda i: (i, 0)),
                  pl.BlockSpec((D, H), lambda i: (0, 0)),
                  pl.BlockSpec((1, H), lambda i: (0, 0)),
                  pl.BlockSpec((H, H), lambda i: (0, 0)),
                  pl.BlockSpec((1, H), lambda i: (0, 0))],
        out_specs=pl.BlockSpec((ROWS, H), lambda i: (i, 0)),
        out_shape=jax.ShapeDtypeStruct((N, H), jnp.float32),
    )(h, pa, pb, w1, b1, w2, b2)


def _layer_body(h_ref, a_ref, w1_ref, b1_ref, w2_ref, b2_ref, o_ref):
    m = h_ref[...] + a_ref[...]
    t = jnp.maximum(
        jnp.dot(m, w1_ref[...], preferred_element_type=jnp.float32)
        + b1_ref[...], 0.0)
    o_ref[...] = jnp.maximum(
        jnp.dot(t, w2_ref[...], preferred_element_type=jnp.float32)
        + b2_ref[...], 0.0)


def _layer(h, a, w1, b1, w2, b2):
    return pl.pallas_call(
        _layer_body,
        grid=(GRID,),
        in_specs=[pl.BlockSpec((ROWS, H), lambda i: (i, 0)),
                  pl.BlockSpec((ROWS, H), lambda i: (i, 0)),
                  pl.BlockSpec((H, H), lambda i: (0, 0)),
                  pl.BlockSpec((1, H), lambda i: (0, 0)),
                  pl.BlockSpec((H, H), lambda i: (0, 0)),
                  pl.BlockSpec((1, H), lambda i: (0, 0))],
        out_specs=pl.BlockSpec((ROWS, H), lambda i: (i, 0)),
        out_shape=jax.ShapeDtypeStruct((N, H), jnp.float32),
    )(h, a, w1, b1, w2, b2)


def _jk_body(h1_ref, h2_ref, h3_ref, wjk_ref, bjk_ref, o_ref):
    hcat = jnp.concatenate([h1_ref[...], h2_ref[...], h3_ref[...]], axis=1)
    o_ref[...] = (jnp.dot(hcat, wjk_ref[...],
                          preferred_element_type=jnp.float32)
                  + bjk_ref[...])


def _jk(h1, h2, h3, wjk, bjk):
    return pl.pallas_call(
        _jk_body,
        grid=(GRID,),
        in_specs=[pl.BlockSpec((ROWS, H), lambda i: (i, 0)),
                  pl.BlockSpec((ROWS, H), lambda i: (i, 0)),
                  pl.BlockSpec((ROWS, H), lambda i: (i, 0)),
                  pl.BlockSpec((3 * H, H), lambda i: (0, 0)),
                  pl.BlockSpec((1, H), lambda i: (0, 0))],
        out_specs=pl.BlockSpec((ROWS, H), lambda i: (i, 0)),
        out_shape=jax.ShapeDtypeStruct((N, H), jnp.float32),
    )(h1, h2, h3, wjk, bjk)


def _cls_body(p_ref, wc1_ref, bc1_ref, g_ref, be_ref, wc2_ref, bc2_ref,
              o_ref):
    pooled = p_ref[0] + p_ref[1]
    z = (jnp.dot(pooled, wc1_ref[...], preferred_element_type=jnp.float32)
         + bc1_ref[...])
    rmask = lax.broadcasted_iota(jnp.int32, (PR, H), 0) < G
    zm = jnp.where(rmask, z, 0.0)
    mean = jnp.sum(zm, axis=0, keepdims=True) / G
    zc = jnp.where(rmask, z - mean, 0.0)
    var = jnp.sum(zc * zc, axis=0, keepdims=True) / G
    zn = (z - mean) / jnp.sqrt(var + 1e-5) * g_ref[...] + be_ref[...]
    zn = jnp.maximum(zn, 0.0)
    o_ref[...] = (jnp.dot(zn, wc2_ref[...], preferred_element_type=jnp.float32)
                  + bc2_ref[...])


def _cls(p, wc1, bc1, gamma, beta, wc2, bc2):
    return pl.pallas_call(
        _cls_body,
        in_specs=[pl.BlockSpec((NC, PR, H), lambda: (0, 0, 0)),
                  pl.BlockSpec((H, H), lambda: (0, 0)),
                  pl.BlockSpec((1, H), lambda: (0, 0)),
                  pl.BlockSpec((1, H), lambda: (0, 0)),
                  pl.BlockSpec((1, H), lambda: (0, 0)),
                  pl.BlockSpec((H, H), lambda: (0, 0)),
                  pl.BlockSpec((1, H), lambda: (0, 0))],
        out_specs=pl.BlockSpec((PR, H), lambda: (0, 0)),
        out_shape=jax.ShapeDtypeStruct((PR, H), jnp.float32),
    )(p, wc1, bc1, gamma, beta, wc2, bc2)


def _edge_slabs(src, dst):
    order = jnp.argsort(dst, stable=True)
    sd = dst[order].reshape(NW, EPS)
    ss = src[order].reshape(NW, EPS)

    firsts = sd[:, 0]
    crossing = jnp.concatenate(
        [jnp.zeros((1,), bool), firsts[1:] == sd[:-1, -1]])
    tid = jnp.arange(NW, dtype=sd.dtype)
    sdr = jnp.where(crossing[:, None] & (sd == firsts[:, None]),
                    PRIV + tid[:, None], sd)
    bd = jnp.where(crossing, firsts, SINK)

    j = jnp.arange(EPS, dtype=jnp.int32)
    change = jnp.concatenate(
        [jnp.ones((NW, 1), bool), sd[:, 1:] != sd[:, :-1]], axis=1)
    fo = lax.cummax(jnp.where(change, j[None, :], 0), axis=1)

    def step(b, _):
        cand = jnp.minimum(b + NCHUNK, EPS)
        fc = jnp.where(
            cand >= EPS, EPS,
            jnp.take_along_axis(
                fo, jnp.minimum(cand, EPS - 1)[:, None], axis=1)[:, 0])
        nb = jnp.maximum(fc, jnp.minimum(b + MINP, EPS))
        return nb, nb

    _, bs = lax.scan(step, jnp.zeros((NW,), jnp.int32), None,
                     length=CHUNK - 1, unroll=8)
    bsT = jnp.concatenate([jnp.zeros((1, NW), jnp.int32), bs]).T
    tid2 = jnp.broadcast_to(tid[:, None], bsT.shape)
    mark = jnp.zeros((NW, EPS), jnp.int32).at[tid2, bsT].add(
        1, mode='drop')
    seg = jnp.cumsum(mark, axis=1) - 1
    start = lax.cummax(jnp.where(mark > 0, j[None, :], 0), axis=1)
    off = j[None, :] - start
    pos = (tid[:, None] * EPT + off * CHUNK + seg).reshape(-1)

    gidx = jnp.arange(NW * EPT, dtype=sd.dtype)
    pad_rows = PRIV + NW + 32 + gidx % 64
    grid_d = pad_rows.at[pos].add(
        sdr.reshape(-1) - (PRIV + NW + 32 + pos % 64))
    pad_src = gidx % N
    grid_s = pad_src.at[pos].add(ss.reshape(-1) - pos % N)
    srcp = grid_s.reshape(NW, NCHUNK, CHUNK)
    dstp = grid_d.reshape(NW, NCHUNK, CHUNK)
    return srcp, dstp, bd


def kernel(x, W1_0, b1_0, W2_0, b2_0, W1_1, b1_1, W2_1, b2_1,
           W1_2, b1_2, W2_2, b2_2, Wjk, bjk, Wc1, bc1, gamma, beta,
           Wc2, bc2, edge_index, batch):
    srcp, dstp, bd = _edge_slabs(edge_index[0], edge_index[1])

    def merge(p):
        a = p[0] + p[1]
        priv = lax.slice(a, (PRIV, 0), (PRIV + NW, a.shape[1]))
        return a.at[bd].add(priv)

    nodep = jnp.pad(jnp.arange(N, dtype=jnp.int32),
                    (0, NPAD - N)).reshape(NW, PCHUNK, CHUNK)
    batchp = jnp.pad(batch, (0, NPAD - N),
                     constant_values=PSINK).reshape(NW, PCHUNK, CHUNK)
    r = lambda b: b.reshape(1, -1)

    xa = lax.slice(x, (0, 0), (N, H))
    xb = lax.slice(x, (0, H), (N, D))
    a0a = merge(_sc_agg(xa, srcp, dstp))
    a0b = merge(_sc_agg(xb, srcp, dstp))
    h1 = _layer0(x, a0a, a0b, W1_0, r(b1_0), W2_0, r(b2_0))
    a1 = merge(_sc_agg(h1, srcp, dstp))
    h2 = _layer(h1, a1, W1_1, r(b1_1), W2_1, r(b2_1))
    a2 = merge(_sc_agg(h2, srcp, dstp))
    h3 = _layer(h2, a2, W1_2, r(b1_2), W2_2, r(b2_2))
    hjk = _jk(h1, h2, h3, Wjk, r(bjk))
    p = _sc_pool(hjk, nodep, batchp)
    out = _cls(p, Wc1, r(bc1), r(gamma), r(beta), Wc2, r(bc2))
    return out[:G]
